# baseline (device time: 126526 ns/iter reference)
import jax
import jax.numpy as jnp
from jax import lax
from jax.experimental import pallas as pl
from jax.experimental.pallas import tpu as pltpu

S = 1024
D = 2048
DC_SH = 128
H = 16
DH = 128
DR = 32
SCALE = (DH + DR) ** -0.5
S4 = S // 4
NR = 4
SQ = S4 // NR
HG = 4
NG = H // HG
GW = HG * DH

F32 = jnp.float32


def _dot(a, b):
    return jnp.dot(a, b, preferred_element_type=F32)


def _dot_t(a, b):
    return lax.dot_general(a, b, (((1,), (1,)), ((), ())),
                           preferred_element_type=F32)


def _pre_body(x_ref, wdkv_ref, wkr_ref, c_ref, kr_ref, xm_ref):
    my_x = lax.axis_index("x")
    my_y = lax.axis_index("y")
    p = my_x * 2 + my_y
    x = x_ref[0]
    c_ref[...] = _dot(x, wdkv_ref[...])
    kr_ref[...] = _dot(x, wkr_ref[...])
    xm_ref[...] = x_ref[0, pl.ds(p * S4, S4), :]


def _body(xm_ref, c_mine_ref, kr_ref, wuk_ref, wuv_ref, wuk_g_ref,
          wuv_g_ref, wq_ref, wqr_ref, wo_ref, out_ref,
          c_oth_s, wuk_o, wuv_o, k_cache, v_cache, out_acc,
          w_send, w_recv, copy_sem, ag_send, ag_recv):
    my_x = lax.axis_index("x")
    my_y = lax.axis_index("y")
    p = my_x * 2 + my_y
    xnbr = (1 - my_x, my_y)
    peers = [(1 - my_x, my_y), (my_x, 1 - my_y), (1 - my_x, 1 - my_y)]
    peer_slots = [(1 - my_x) * 2 + my_y, my_x * 2 + (1 - my_y),
                  (1 - my_x) * 2 + (1 - my_y)]
    r = pl.program_id(0)
    g = pl.program_id(1)
    barrier = pltpu.get_barrier_semaphore()

    def _wrdma(i, src, dst):
        return pltpu.make_async_remote_copy(
            src_ref=src, dst_ref=dst,
            send_sem=w_send.at[i], recv_sem=w_recv.at[i],
            device_id=xnbr, device_id_type=pl.DeviceIdType.MESH)

    def _w_descs():
        descs = [_wrdma(0, c_mine_ref, c_oth_s)]
        for gg in range(NG):
            descs.append(_wrdma(1 + 2 * gg,
                                wuk_ref.at[:, gg * GW:(gg + 1) * GW],
                                wuk_o.at[gg]))
            descs.append(_wrdma(2 + 2 * gg,
                                wuv_ref.at[:, gg * GW:(gg + 1) * GW],
                                wuv_o.at[gg]))
        return descs

    @pl.when((r == 0) & (g == 0))
    def _():
        for peer in peers:
            pl.semaphore_signal(barrier, inc=1, device_id=peer,
                                device_id_type=pl.DeviceIdType.MESH)
        pl.semaphore_wait(barrier, 3)
        for d in _w_descs():
            d.start()

    x_q = xm_ref[pl.ds(r * SQ, SQ), :]
    q = _dot(x_q, wq_ref[...])
    qr = _dot(x_q, wqr_ref[...])

    @pl.when((r == 0) & (g == 0))
    def _():
        _wrdma(0, c_mine_ref, c_oth_s).wait_recv()

    for gg in range(NG):
        @pl.when((r == 0) & (g == gg))
        def _(gg=gg):
            _wrdma(1 + 2 * gg, wuk_ref.at[:, gg * GW:(gg + 1) * GW],
                   wuk_o.at[gg]).wait_recv()
            _wrdma(2 + 2 * gg, wuv_ref.at[:, gg * GW:(gg + 1) * GW],
                   wuv_o.at[gg]).wait_recv()
            c_mine = c_mine_ref[...]
            c_oth = c_oth_s[...]
            k_cache[gg] = (_dot(c_mine, wuk_g_ref[...])
                           + _dot(c_oth, wuk_o[gg]))
            v_cache[gg] = (_dot(c_mine, wuv_g_ref[...])
                           + _dot(c_oth, wuv_o[gg]))

    k_g = k_cache[g]
    v_g = v_cache[g]
    kr = kr_ref[...]

    o_cols = []
    for i in range(HG):
        s = _dot_t(q[:, i * DH:(i + 1) * DH], k_g[:, i * DH:(i + 1) * DH])
        s = s + _dot_t(qr[:, i * DR:(i + 1) * DR], kr)
        s = s * SCALE
        m = jnp.max(s, axis=1, keepdims=True)
        pr = jnp.exp(s - m)
        pr = pr / jnp.sum(pr, axis=1, keepdims=True)
        o_cols.append(_dot(pr, v_g[:, i * DH:(i + 1) * DH]))
    proj = _dot(jnp.concatenate(o_cols, axis=1), wo_ref[...])

    @pl.when(g == 0)
    def _():
        out_acc[r] = proj

    @pl.when(g != 0)
    def _():
        out_acc[r] = out_acc[r] + proj

    for rr in range(NR):
        @pl.when((r == rr) & (g == NG - 1))
        def _(rr=rr):
            my_rows = out_ref.at[0, pl.ds(p * S4 + rr * SQ, SQ), :]
            cp = pltpu.make_async_copy(out_acc.at[rr], my_rows, copy_sem)
            cp.start()
            cp.wait()
            for i, peer in enumerate(peers):
                pltpu.make_async_remote_copy(
                    src_ref=my_rows, dst_ref=my_rows,
                    send_sem=ag_send.at[rr * 3 + i],
                    recv_sem=ag_recv.at[p * NR + rr],
                    device_id=peer,
                    device_id_type=pl.DeviceIdType.MESH).start()

    @pl.when((r == NR - 1) & (g == NG - 1))
    def _():
        for d in _w_descs():
            d.wait_send()
        for rr in range(NR):
            my_rows = out_ref.at[0, pl.ds(p * S4 + rr * SQ, SQ), :]
            for i, peer in enumerate(peers):
                pltpu.make_async_remote_copy(
                    src_ref=my_rows, dst_ref=my_rows,
                    send_sem=ag_send.at[rr * 3 + i],
                    recv_sem=ag_recv.at[p * NR + rr],
                    device_id=peer,
                    device_id_type=pl.DeviceIdType.MESH).wait_send()
        for rr in range(NR):
            for i, qp in enumerate(peer_slots):
                qrows = out_ref.at[0, pl.ds(qp * S4 + rr * SQ, SQ), :]
                pltpu.make_async_remote_copy(
                    src_ref=qrows, dst_ref=qrows,
                    send_sem=ag_send.at[rr * 3 + i],
                    recv_sem=ag_recv.at[qp * NR + rr],
                    device_id=(my_x, my_y),
                    device_id_type=pl.DeviceIdType.MESH).wait_recv()


def kernel(x, Wdkv, Wuk, Wuv, Wq, Wqr, Wkr, Wo):
    c_mine, kr, xm = pl.pallas_call(
        _pre_body,
        out_shape=(
            jax.ShapeDtypeStruct((S, DC_SH), F32),
            jax.ShapeDtypeStruct((S, DR), F32),
            jax.ShapeDtypeStruct((S4, D), F32),
        ),
        in_specs=[pl.BlockSpec(memory_space=pltpu.VMEM)] * 3,
        out_specs=(pl.BlockSpec(memory_space=pltpu.VMEM),) * 3,
    )(x, Wdkv, Wkr)

    y = pl.pallas_call(
        _body,
        grid=(NR, NG),
        out_shape=jax.ShapeDtypeStruct((1, S, D), F32),
        in_specs=[
            pl.BlockSpec((S4, D), lambda r, g: (0, 0)),
            pl.BlockSpec((S, DC_SH), lambda r, g: (0, 0)),
            pl.BlockSpec((S, DR), lambda r, g: (0, 0)),
            pl.BlockSpec((DC_SH, D), lambda r, g: (0, 0)),
            pl.BlockSpec((DC_SH, D), lambda r, g: (0, 0)),
            pl.BlockSpec((DC_SH, GW), lambda r, g: (0, g)),
            pl.BlockSpec((DC_SH, GW), lambda r, g: (0, g)),
            pl.BlockSpec((D, GW), lambda r, g: (0, g)),
            pl.BlockSpec((D, HG * DR), lambda r, g: (0, g)),
            pl.BlockSpec((GW, D), lambda r, g: (g, 0)),
        ],
        out_specs=pl.BlockSpec(memory_space=pl.ANY),
        scratch_shapes=[
            pltpu.VMEM((S, DC_SH), F32),
            pltpu.VMEM((NG, DC_SH, GW), F32),
            pltpu.VMEM((NG, DC_SH, GW), F32),
            pltpu.VMEM((NG, S, GW), F32),
            pltpu.VMEM((NG, S, GW), F32),
            pltpu.VMEM((NR, SQ, D), F32),
            pltpu.SemaphoreType.DMA((9,)),
            pltpu.SemaphoreType.DMA((9,)),
            pltpu.SemaphoreType.DMA,
            pltpu.SemaphoreType.DMA((NR * 3,)),
            pltpu.SemaphoreType.DMA((4 * NR,)),
        ],
        compiler_params=pltpu.CompilerParams(
            collective_id=0, vmem_limit_bytes=60 * 1024 * 1024,
        ),
    )(xm, c_mine, kr, Wuk, Wuv, Wuk, Wuv, Wq, Wqr, Wo)

    return y


# device time: 111980 ns/iter; 1.1299x vs baseline; 1.1299x over previous
import jax
import jax.numpy as jnp
from jax import lax
from jax.experimental import pallas as pl
from jax.experimental.pallas import tpu as pltpu

S = 1024
D = 2048
DC_SH = 128
H = 16
DH = 128
DR = 32
SCALE = (DH + DR) ** -0.5
S4 = S // 4
NR = 2
SQ = S4 // NR
HG = 4
NG = H // HG
GW = HG * DH

F32 = jnp.float32


def _dot(a, b):
    return jnp.dot(a, b, preferred_element_type=F32)


def _dot_t(a, b):
    return lax.dot_general(a, b, (((1,), (1,)), ((), ())),
                           preferred_element_type=F32)


def _pre_body(x_ref, wdkv_ref, wkr_ref, c_ref, kr_ref, xm_ref):
    my_x = lax.axis_index("x")
    my_y = lax.axis_index("y")
    p = my_x * 2 + my_y
    x = x_ref[0]
    c_ref[...] = _dot(x, wdkv_ref[...])
    kr_ref[...] = _dot(x, wkr_ref[...])
    xm_ref[...] = x_ref[0, pl.ds(p * S4, S4), :]


def _body(xm_ref, c_mine_ref, kr_ref, wuk_ref, wuv_ref, wuk_g_ref,
          wuv_g_ref, wq_ref, wqr_ref, wo_ref, out_ref,
          c_oth_s, wuk_o, wuv_o, k_cache, v_cache, out_acc,
          w_send, w_recv, copy_sem, ag_send, ag_recv):
    my_x = lax.axis_index("x")
    my_y = lax.axis_index("y")
    p = my_x * 2 + my_y
    xnbr = (1 - my_x, my_y)
    peers = [(1 - my_x, my_y), (my_x, 1 - my_y), (1 - my_x, 1 - my_y)]
    peer_slots = [(1 - my_x) * 2 + my_y, my_x * 2 + (1 - my_y),
                  (1 - my_x) * 2 + (1 - my_y)]
    r = pl.program_id(0)
    g = pl.program_id(1)
    barrier = pltpu.get_barrier_semaphore()

    def _wrdma(i, src, dst):
        return pltpu.make_async_remote_copy(
            src_ref=src, dst_ref=dst,
            send_sem=w_send.at[i], recv_sem=w_recv.at[i],
            device_id=xnbr, device_id_type=pl.DeviceIdType.MESH)

    def _w_descs():
        descs = [_wrdma(0, c_mine_ref, c_oth_s)]
        for gg in range(NG):
            descs.append(_wrdma(1 + 2 * gg,
                                wuk_ref.at[:, gg * GW:(gg + 1) * GW],
                                wuk_o.at[gg]))
            descs.append(_wrdma(2 + 2 * gg,
                                wuv_ref.at[:, gg * GW:(gg + 1) * GW],
                                wuv_o.at[gg]))
        return descs

    @pl.when((r == 0) & (g == 0))
    def _():
        for peer in peers:
            pl.semaphore_signal(barrier, inc=1, device_id=peer,
                                device_id_type=pl.DeviceIdType.MESH)
        pl.semaphore_wait(barrier, 3)
        for d in _w_descs():
            d.start()

    x_q = xm_ref[pl.ds(r * SQ, SQ), :]
    q = _dot(x_q, wq_ref[...])
    qr = _dot(x_q, wqr_ref[...])

    @pl.when((r == 0) & (g == 0))
    def _():
        _wrdma(0, c_mine_ref, c_oth_s).wait_recv()

    for gg in range(NG):
        @pl.when((r == 0) & (g == gg))
        def _(gg=gg):
            _wrdma(1 + 2 * gg, wuk_ref.at[:, gg * GW:(gg + 1) * GW],
                   wuk_o.at[gg]).wait_recv()
            _wrdma(2 + 2 * gg, wuv_ref.at[:, gg * GW:(gg + 1) * GW],
                   wuv_o.at[gg]).wait_recv()
            c_mine = c_mine_ref[...]
            c_oth = c_oth_s[...]
            k_cache[gg] = (_dot(c_mine, wuk_g_ref[...])
                           + _dot(c_oth, wuk_o[gg]))
            v_cache[gg] = (_dot(c_mine, wuv_g_ref[...])
                           + _dot(c_oth, wuv_o[gg]))

    k_g = k_cache[g]
    v_g = v_cache[g]
    kr = kr_ref[...]

    o_cols = []
    for i in range(HG):
        s = _dot_t(q[:, i * DH:(i + 1) * DH], k_g[:, i * DH:(i + 1) * DH])
        s = s + _dot_t(qr[:, i * DR:(i + 1) * DR], kr)
        s = s * SCALE
        m = jnp.max(s, axis=1, keepdims=True)
        pr = jnp.exp(s - m)
        pr = pr / jnp.sum(pr, axis=1, keepdims=True)
        o_cols.append(_dot(pr, v_g[:, i * DH:(i + 1) * DH]))
    proj = _dot(jnp.concatenate(o_cols, axis=1), wo_ref[...])

    @pl.when(g == 0)
    def _():
        out_acc[r] = proj

    @pl.when(g != 0)
    def _():
        out_acc[r] = out_acc[r] + proj

    for rr in range(NR):
        @pl.when((r == rr) & (g == NG - 1))
        def _(rr=rr):
            my_rows = out_ref.at[0, pl.ds(p * S4 + rr * SQ, SQ), :]
            cp = pltpu.make_async_copy(out_acc.at[rr], my_rows, copy_sem)
            cp.start()
            cp.wait()
            for i, peer in enumerate(peers):
                pltpu.make_async_remote_copy(
                    src_ref=my_rows, dst_ref=my_rows,
                    send_sem=ag_send.at[rr * 3 + i],
                    recv_sem=ag_recv.at[p * NR + rr],
                    device_id=peer,
                    device_id_type=pl.DeviceIdType.MESH).start()

    @pl.when((r == NR - 1) & (g == NG - 1))
    def _():
        for d in _w_descs():
            d.wait_send()
        for rr in range(NR):
            my_rows = out_ref.at[0, pl.ds(p * S4 + rr * SQ, SQ), :]
            for i, peer in enumerate(peers):
                pltpu.make_async_remote_copy(
                    src_ref=my_rows, dst_ref=my_rows,
                    send_sem=ag_send.at[rr * 3 + i],
                    recv_sem=ag_recv.at[p * NR + rr],
                    device_id=peer,
                    device_id_type=pl.DeviceIdType.MESH).wait_send()
        for rr in range(NR):
            for i, qp in enumerate(peer_slots):
                qrows = out_ref.at[0, pl.ds(qp * S4 + rr * SQ, SQ), :]
                pltpu.make_async_remote_copy(
                    src_ref=qrows, dst_ref=qrows,
                    send_sem=ag_send.at[rr * 3 + i],
                    recv_sem=ag_recv.at[qp * NR + rr],
                    device_id=(my_x, my_y),
                    device_id_type=pl.DeviceIdType.MESH).wait_recv()


def kernel(x, Wdkv, Wuk, Wuv, Wq, Wqr, Wkr, Wo):
    c_mine, kr, xm = pl.pallas_call(
        _pre_body,
        out_shape=(
            jax.ShapeDtypeStruct((S, DC_SH), F32),
            jax.ShapeDtypeStruct((S, DR), F32),
            jax.ShapeDtypeStruct((S4, D), F32),
        ),
        in_specs=[pl.BlockSpec(memory_space=pltpu.VMEM)] * 3,
        out_specs=(pl.BlockSpec(memory_space=pltpu.VMEM),) * 3,
    )(x, Wdkv, Wkr)

    y = pl.pallas_call(
        _body,
        grid=(NR, NG),
        out_shape=jax.ShapeDtypeStruct((1, S, D), F32),
        in_specs=[
            pl.BlockSpec((S4, D), lambda r, g: (0, 0)),
            pl.BlockSpec((S, DC_SH), lambda r, g: (0, 0)),
            pl.BlockSpec((S, DR), lambda r, g: (0, 0)),
            pl.BlockSpec((DC_SH, D), lambda r, g: (0, 0)),
            pl.BlockSpec((DC_SH, D), lambda r, g: (0, 0)),
            pl.BlockSpec((DC_SH, GW), lambda r, g: (0, g)),
            pl.BlockSpec((DC_SH, GW), lambda r, g: (0, g)),
            pl.BlockSpec((D, GW), lambda r, g: (0, g)),
            pl.BlockSpec((D, HG * DR), lambda r, g: (0, g)),
            pl.BlockSpec((GW, D), lambda r, g: (g, 0)),
        ],
        out_specs=pl.BlockSpec(memory_space=pl.ANY),
        scratch_shapes=[
            pltpu.VMEM((S, DC_SH), F32),
            pltpu.VMEM((NG, DC_SH, GW), F32),
            pltpu.VMEM((NG, DC_SH, GW), F32),
            pltpu.VMEM((NG, S, GW), F32),
            pltpu.VMEM((NG, S, GW), F32),
            pltpu.VMEM((NR, SQ, D), F32),
            pltpu.SemaphoreType.DMA((9,)),
            pltpu.SemaphoreType.DMA((9,)),
            pltpu.SemaphoreType.DMA,
            pltpu.SemaphoreType.DMA((NR * 3,)),
            pltpu.SemaphoreType.DMA((4 * NR,)),
        ],
        compiler_params=pltpu.CompilerParams(
            collective_id=0, vmem_limit_bytes=60 * 1024 * 1024,
        ),
    )(xm, c_mine, kr, Wuk, Wuv, Wuk, Wuv, Wq, Wqr, Wo)

    return y


# device time: 110870 ns/iter; 1.1412x vs baseline; 1.0100x over previous
import jax
import jax.numpy as jnp
from jax import lax
from jax.experimental import pallas as pl
from jax.experimental.pallas import tpu as pltpu

S = 1024
D = 2048
DC_SH = 128
H = 16
DH = 128
DR = 32
SCALE = (DH + DR) ** -0.5
S4 = S // 4
NR = 2
SQ = S4 // NR
HG = 4
NG = H // HG
GW = HG * DH

F32 = jnp.float32
BF16 = jnp.bfloat16


def _dot(a, b):
    return jnp.dot(a, b, preferred_element_type=F32)


def _dot_t(a, b):
    return lax.dot_general(a, b, (((1,), (1,)), ((), ())),
                           preferred_element_type=F32)


def _pre_body(x_ref, wdkv_ref, wkr_ref, c_ref, kr_ref, xm_ref):
    my_x = lax.axis_index("x")
    my_y = lax.axis_index("y")
    p = my_x * 2 + my_y
    x = x_ref[0]
    c_ref[...] = _dot(x, wdkv_ref[...])
    kr_ref[...] = _dot(x, wkr_ref[...])
    xm_ref[...] = x_ref[0, pl.ds(p * S4, S4), :]


def _body(xm_ref, c_mine_ref, kr_ref, wuk_ref, wuv_ref, wuk_g_ref,
          wuv_g_ref, wq_ref, wqr_ref, wo_ref, out_ref,
          c_oth_s, c_mine_bf, c_oth_bf, wuk_o, wuv_o, k_cache, v_cache,
          out_acc, w_send, w_recv, copy_sem, ag_send, ag_recv):
    my_x = lax.axis_index("x")
    my_y = lax.axis_index("y")
    p = my_x * 2 + my_y
    xnbr = (1 - my_x, my_y)
    peers = [(1 - my_x, my_y), (my_x, 1 - my_y), (1 - my_x, 1 - my_y)]
    peer_slots = [(1 - my_x) * 2 + my_y, my_x * 2 + (1 - my_y),
                  (1 - my_x) * 2 + (1 - my_y)]
    r = pl.program_id(0)
    g = pl.program_id(1)
    barrier = pltpu.get_barrier_semaphore()

    def _wrdma(i, src, dst):
        return pltpu.make_async_remote_copy(
            src_ref=src, dst_ref=dst,
            send_sem=w_send.at[i], recv_sem=w_recv.at[i],
            device_id=xnbr, device_id_type=pl.DeviceIdType.MESH)

    def _w_descs():
        descs = [_wrdma(0, c_mine_ref, c_oth_s)]
        for gg in range(NG):
            descs.append(_wrdma(1 + 2 * gg,
                                wuk_ref.at[:, gg * GW:(gg + 1) * GW],
                                wuk_o.at[gg]))
            descs.append(_wrdma(2 + 2 * gg,
                                wuv_ref.at[:, gg * GW:(gg + 1) * GW],
                                wuv_o.at[gg]))
        return descs

    @pl.when((r == 0) & (g == 0))
    def _():
        for peer in peers:
            pl.semaphore_signal(barrier, inc=1, device_id=peer,
                                device_id_type=pl.DeviceIdType.MESH)
        pl.semaphore_wait(barrier, 3)
        for d in _w_descs():
            d.start()

    x_q = xm_ref[pl.ds(r * SQ, SQ), :]
    q = _dot(x_q, wq_ref[...]) * SCALE
    qr = _dot(x_q, wqr_ref[...]) * SCALE

    @pl.when((r == 0) & (g == 0))
    def _():
        c_mine_bf[...] = c_mine_ref[...].astype(BF16)
        _wrdma(0, c_mine_ref, c_oth_s).wait_recv()
        c_oth_bf[...] = c_oth_s[...].astype(BF16)

    for gg in range(NG):
        @pl.when((r == 0) & (g == gg))
        def _(gg=gg):
            _wrdma(1 + 2 * gg, wuk_ref.at[:, gg * GW:(gg + 1) * GW],
                   wuk_o.at[gg]).wait_recv()
            _wrdma(2 + 2 * gg, wuv_ref.at[:, gg * GW:(gg + 1) * GW],
                   wuv_o.at[gg]).wait_recv()
            cm = c_mine_bf[...]
            co = c_oth_bf[...]
            k_cache[gg] = (_dot(cm, wuk_g_ref[...].astype(BF16))
                           + _dot(co, wuk_o[gg].astype(BF16))).astype(BF16)
            v_cache[gg] = (_dot(cm, wuv_g_ref[...].astype(BF16))
                           + _dot(co, wuv_o[gg].astype(BF16))).astype(BF16)

    k_g = k_cache[g]
    v_g = v_cache[g]
    kr = kr_ref[...].astype(BF16)

    o_cols = []
    for i in range(HG):
        s = _dot_t(q[:, i * DH:(i + 1) * DH].astype(BF16),
                   k_g[:, i * DH:(i + 1) * DH])
        s = s + _dot_t(qr[:, i * DR:(i + 1) * DR].astype(BF16), kr)
        e = jnp.exp(s)
        denom = jnp.sum(e, axis=1, keepdims=True)
        o_cols.append(
            _dot(e.astype(BF16), v_g[:, i * DH:(i + 1) * DH]) / denom)
    proj = _dot(jnp.concatenate(o_cols, axis=1), wo_ref[...])

    @pl.when(g == 0)
    def _():
        out_acc[r] = proj

    @pl.when(g != 0)
    def _():
        out_acc[r] = out_acc[r] + proj

    for rr in range(NR):
        @pl.when((r == rr) & (g == NG - 1))
        def _(rr=rr):
            my_rows = out_ref.at[0, pl.ds(p * S4 + rr * SQ, SQ), :]
            cp = pltpu.make_async_copy(out_acc.at[rr], my_rows, copy_sem)
            cp.start()
            cp.wait()
            for i, peer in enumerate(peers):
                pltpu.make_async_remote_copy(
                    src_ref=my_rows, dst_ref=my_rows,
                    send_sem=ag_send.at[rr * 3 + i],
                    recv_sem=ag_recv.at[p * NR + rr],
                    device_id=peer,
                    device_id_type=pl.DeviceIdType.MESH).start()

    @pl.when((r == NR - 1) & (g == NG - 1))
    def _():
        for d in _w_descs():
            d.wait_send()
        for rr in range(NR):
            my_rows = out_ref.at[0, pl.ds(p * S4 + rr * SQ, SQ), :]
            for i, peer in enumerate(peers):
                pltpu.make_async_remote_copy(
                    src_ref=my_rows, dst_ref=my_rows,
                    send_sem=ag_send.at[rr * 3 + i],
                    recv_sem=ag_recv.at[p * NR + rr],
                    device_id=peer,
                    device_id_type=pl.DeviceIdType.MESH).wait_send()
        for rr in range(NR):
            for i, qp in enumerate(peer_slots):
                qrows = out_ref.at[0, pl.ds(qp * S4 + rr * SQ, SQ), :]
                pltpu.make_async_remote_copy(
                    src_ref=qrows, dst_ref=qrows,
                    send_sem=ag_send.at[rr * 3 + i],
                    recv_sem=ag_recv.at[qp * NR + rr],
                    device_id=(my_x, my_y),
                    device_id_type=pl.DeviceIdType.MESH).wait_recv()


def kernel(x, Wdkv, Wuk, Wuv, Wq, Wqr, Wkr, Wo):
    c_mine, kr, xm = pl.pallas_call(
        _pre_body,
        out_shape=(
            jax.ShapeDtypeStruct((S, DC_SH), F32),
            jax.ShapeDtypeStruct((S, DR), F32),
            jax.ShapeDtypeStruct((S4, D), F32),
        ),
        in_specs=[pl.BlockSpec(memory_space=pltpu.VMEM)] * 3,
        out_specs=(pl.BlockSpec(memory_space=pltpu.VMEM),) * 3,
    )(x, Wdkv, Wkr)

    y = pl.pallas_call(
        _body,
        grid=(NR, NG),
        out_shape=jax.ShapeDtypeStruct((1, S, D), F32),
        in_specs=[
            pl.BlockSpec((S4, D), lambda r, g: (0, 0)),
            pl.BlockSpec((S, DC_SH), lambda r, g: (0, 0)),
            pl.BlockSpec((S, DR), lambda r, g: (0, 0)),
            pl.BlockSpec((DC_SH, D), lambda r, g: (0, 0)),
            pl.BlockSpec((DC_SH, D), lambda r, g: (0, 0)),
            pl.BlockSpec((DC_SH, GW), lambda r, g: (0, g)),
            pl.BlockSpec((DC_SH, GW), lambda r, g: (0, g)),
            pl.BlockSpec((D, GW), lambda r, g: (0, g)),
            pl.BlockSpec((D, HG * DR), lambda r, g: (0, g)),
            pl.BlockSpec((GW, D), lambda r, g: (g, 0)),
        ],
        out_specs=pl.BlockSpec(memory_space=pl.ANY),
        scratch_shapes=[
            pltpu.VMEM((S, DC_SH), F32),
            pltpu.VMEM((S, DC_SH), BF16),
            pltpu.VMEM((S, DC_SH), BF16),
            pltpu.VMEM((NG, DC_SH, GW), F32),
            pltpu.VMEM((NG, DC_SH, GW), F32),
            pltpu.VMEM((NG, S, GW), BF16),
            pltpu.VMEM((NG, S, GW), BF16),
            pltpu.VMEM((NR, SQ, D), F32),
            pltpu.SemaphoreType.DMA((9,)),
            pltpu.SemaphoreType.DMA((9,)),
            pltpu.SemaphoreType.DMA,
            pltpu.SemaphoreType.DMA((NR * 3,)),
            pltpu.SemaphoreType.DMA((4 * NR,)),
        ],
        compiler_params=pltpu.CompilerParams(
            collective_id=0, vmem_limit_bytes=60 * 1024 * 1024,
        ),
    )(xm, c_mine, kr, Wuk, Wuv, Wuk, Wuv, Wq, Wqr, Wo)

    return y


# device time: 79991 ns/iter; 1.5818x vs baseline; 1.3860x over previous
import jax
import jax.numpy as jnp
from jax import lax
from jax.experimental import pallas as pl
from jax.experimental.pallas import tpu as pltpu

S = 1024
D = 2048
DC_SH = 128
H = 16
DH = 128
DR = 32
SCALE = (DH + DR) ** -0.5
S4 = S // 4
NR = 2
SQ = S4 // NR
HG = 4
NG = H // HG
GW = HG * DH

F32 = jnp.float32
BF16 = jnp.bfloat16


def _dot(a, b):
    return jnp.dot(a, b, preferred_element_type=F32)


def _dot_t(a, b):
    return lax.dot_general(a, b, (((1,), (1,)), ((), ())),
                           preferred_element_type=F32)


def _body(x_ref, wdkv_ref, wkr_ref, wuk_ref, wuv_ref,
          wq_ref, wqr_ref, wo_ref, out_ref,
          c_mine_bf, kr_s, wuk_bf, wuv_bf,
          c_oth_bf, wuk_o, wuv_o, k_cache, v_cache,
          out_acc, stage_bf, recv_bf, tmp_f32,
          w_send, w_recv, copy_sem, ag_send, ag_recv):
    my_x = lax.axis_index("x")
    my_y = lax.axis_index("y")
    p = my_x * 2 + my_y
    xnbr = (1 - my_x, my_y)
    peers = [(1 - my_x, my_y), (my_x, 1 - my_y), (1 - my_x, 1 - my_y)]
    peer_slots = [(1 - my_x) * 2 + my_y, my_x * 2 + (1 - my_y),
                  (1 - my_x) * 2 + (1 - my_y)]
    r = pl.program_id(0)
    g = pl.program_id(1)
    barrier = pltpu.get_barrier_semaphore()

    def _wrdma(i, src, dst):
        return pltpu.make_async_remote_copy(
            src_ref=src, dst_ref=dst,
            send_sem=w_send.at[i], recv_sem=w_recv.at[i],
            device_id=xnbr, device_id_type=pl.DeviceIdType.MESH)

    def _w_descs():
        descs = [_wrdma(0, c_mine_bf, c_oth_bf)]
        for gg in range(NG):
            descs.append(_wrdma(1 + 2 * gg, wuk_bf.at[gg], wuk_o.at[gg]))
            descs.append(_wrdma(2 + 2 * gg, wuv_bf.at[gg], wuv_o.at[gg]))
        return descs

    @pl.when((r == 0) & (g == 0))
    def _():
        for peer in peers:
            pl.semaphore_signal(barrier, inc=1, device_id=peer,
                                device_id_type=pl.DeviceIdType.MESH)
        pl.semaphore_wait(barrier, 3)
        c_mine_bf[...] = _dot(x_ref[0], wdkv_ref[...]).astype(BF16)
        for gg in range(NG):
            wuk_bf[gg] = wuk_ref[:, gg * GW:(gg + 1) * GW].astype(BF16)
            wuv_bf[gg] = wuv_ref[:, gg * GW:(gg + 1) * GW].astype(BF16)
        for d in _w_descs():
            d.start()
        kr_s[...] = _dot(x_ref[0], wkr_ref[...])

    x_q = x_ref[0, pl.ds(p * S4 + r * SQ, SQ), :]
    q = _dot(x_q, wq_ref[...]) * SCALE
    qr = _dot(x_q, wqr_ref[...]) * SCALE

    @pl.when((r == 0) & (g == 0))
    def _():
        _wrdma(0, c_mine_bf, c_oth_bf).wait_recv()

    for gg in range(NG):
        @pl.when((r == 0) & (g == gg))
        def _(gg=gg):
            _wrdma(1 + 2 * gg, wuk_bf.at[gg], wuk_o.at[gg]).wait_recv()
            _wrdma(2 + 2 * gg, wuv_bf.at[gg], wuv_o.at[gg]).wait_recv()
            cm = c_mine_bf[...]
            co = c_oth_bf[...]
            k_cache[gg] = (_dot(cm, wuk_bf[gg])
                           + _dot(co, wuk_o[gg])).astype(BF16)
            v_cache[gg] = (_dot(cm, wuv_bf[gg])
                           + _dot(co, wuv_o[gg])).astype(BF16)

    k_g = k_cache[g]
    v_g = v_cache[g]
    kr = kr_s[...].astype(BF16)

    o_cols = []
    for i in range(HG):
        s = _dot_t(q[:, i * DH:(i + 1) * DH].astype(BF16),
                   k_g[:, i * DH:(i + 1) * DH])
        s = s + _dot_t(qr[:, i * DR:(i + 1) * DR].astype(BF16), kr)
        e = jnp.exp(s)
        denom = jnp.sum(e, axis=1, keepdims=True)
        o_cols.append(
            _dot(e.astype(BF16), v_g[:, i * DH:(i + 1) * DH]) / denom)
    proj = _dot(jnp.concatenate(o_cols, axis=1), wo_ref[...])

    @pl.when(g == 0)
    def _():
        out_acc[r] = proj

    @pl.when(g != 0)
    def _():
        out_acc[r] = out_acc[r] + proj

    for rr in range(NR):
        @pl.when((r == rr) & (g == NG - 1))
        def _(rr=rr):
            my_rows = out_ref.at[0, pl.ds(p * S4 + rr * SQ, SQ), :]
            cp = pltpu.make_async_copy(out_acc.at[rr], my_rows, copy_sem)
            cp.start()
            stage_bf[rr] = out_acc[rr].astype(BF16)
            cp.wait()
            for i, peer in enumerate(peers):
                pltpu.make_async_remote_copy(
                    src_ref=stage_bf.at[rr], dst_ref=recv_bf.at[p, rr],
                    send_sem=ag_send.at[rr * 3 + i],
                    recv_sem=ag_recv.at[p * NR + rr],
                    device_id=peer,
                    device_id_type=pl.DeviceIdType.MESH).start()

    @pl.when((r == NR - 1) & (g == NG - 1))
    def _():
        for d in _w_descs():
            d.wait_send()
        for rr in range(NR):
            for i, peer in enumerate(peers):
                pltpu.make_async_remote_copy(
                    src_ref=stage_bf.at[rr], dst_ref=recv_bf.at[p, rr],
                    send_sem=ag_send.at[rr * 3 + i],
                    recv_sem=ag_recv.at[p * NR + rr],
                    device_id=peer,
                    device_id_type=pl.DeviceIdType.MESH).wait_send()
        for rr in range(NR):
            for i, qp in enumerate(peer_slots):
                pltpu.make_async_remote_copy(
                    src_ref=recv_bf.at[qp, rr], dst_ref=recv_bf.at[qp, rr],
                    send_sem=ag_send.at[rr * 3 + i],
                    recv_sem=ag_recv.at[qp * NR + rr],
                    device_id=(my_x, my_y),
                    device_id_type=pl.DeviceIdType.MESH).wait_recv()
                tmp_f32[...] = recv_bf[qp, rr].astype(F32)
                cp = pltpu.make_async_copy(
                    tmp_f32,
                    out_ref.at[0, pl.ds(qp * S4 + rr * SQ, SQ), :],
                    copy_sem)
                cp.start()
                cp.wait()


def kernel(x, Wdkv, Wuk, Wuv, Wq, Wqr, Wkr, Wo):
    y = pl.pallas_call(
        _body,
        grid=(NR, NG),
        out_shape=jax.ShapeDtypeStruct((1, S, D), F32),
        in_specs=[
            pl.BlockSpec((1, S, D), lambda r, g: (0, 0, 0)),
            pl.BlockSpec((D, DC_SH), lambda r, g: (0, 0)),
            pl.BlockSpec((D, DR), lambda r, g: (0, 0)),
            pl.BlockSpec((DC_SH, D), lambda r, g: (0, 0)),
            pl.BlockSpec((DC_SH, D), lambda r, g: (0, 0)),
            pl.BlockSpec((D, GW), lambda r, g: (0, g)),
            pl.BlockSpec((D, HG * DR), lambda r, g: (0, g)),
            pl.BlockSpec((GW, D), lambda r, g: (g, 0)),
        ],
        out_specs=pl.BlockSpec(memory_space=pl.ANY),
        scratch_shapes=[
            pltpu.VMEM((S, DC_SH), BF16),
            pltpu.VMEM((S, DR), F32),
            pltpu.VMEM((NG, DC_SH, GW), BF16),
            pltpu.VMEM((NG, DC_SH, GW), BF16),
            pltpu.VMEM((S, DC_SH), BF16),
            pltpu.VMEM((NG, DC_SH, GW), BF16),
            pltpu.VMEM((NG, DC_SH, GW), BF16),
            pltpu.VMEM((NG, S, GW), BF16),
            pltpu.VMEM((NG, S, GW), BF16),
            pltpu.VMEM((NR, SQ, D), F32),
            pltpu.VMEM((NR, SQ, D), BF16),
            pltpu.VMEM((4, NR, SQ, D), BF16),
            pltpu.VMEM((SQ, D), F32),
            pltpu.SemaphoreType.DMA((9,)),
            pltpu.SemaphoreType.DMA((9,)),
            pltpu.SemaphoreType.DMA,
            pltpu.SemaphoreType.DMA((NR * 3,)),
            pltpu.SemaphoreType.DMA((4 * NR,)),
        ],
        compiler_params=pltpu.CompilerParams(
            collective_id=0, vmem_limit_bytes=62 * 1024 * 1024,
        ),
    )(x, Wdkv, Wkr, Wuk, Wuv, Wq, Wqr, Wo)

    return y


# device time: 78902 ns/iter; 1.6036x vs baseline; 1.0138x over previous
import jax
import jax.numpy as jnp
from jax import lax
from jax.experimental import pallas as pl
from jax.experimental.pallas import tpu as pltpu

S = 1024
D = 2048
DC_SH = 128
H = 16
DH = 128
DR = 32
SCALE = (DH + DR) ** -0.5
S4 = S // 4
NR = 2
SQ = S4 // NR
HG = 4
NG = H // HG
GW = HG * DH

F32 = jnp.float32
BF16 = jnp.bfloat16


def _dot(a, b):
    return jnp.dot(a, b, preferred_element_type=F32)


def _dot_t(a, b):
    return lax.dot_general(a, b, (((1,), (1,)), ((), ())),
                           preferred_element_type=F32)


def _body(x_ref, wdkv_ref, wkr_ref, wuk_ref, wuv_ref,
          wq_ref, wqr_ref, wo_ref, out_ref,
          c_mine_bf, kr_s, wuk_bf, wuv_bf,
          c_oth_bf, wuk_o, wuv_o, k_cache, v_cache, q_cache, qr_cache,
          out_acc, stage_bf, recv_bf, tmp_f32,
          w_send, w_recv, copy_sem, ag_send, ag_recv):
    my_x = lax.axis_index("x")
    my_y = lax.axis_index("y")
    p = my_x * 2 + my_y
    xnbr = (1 - my_x, my_y)
    peers = [(1 - my_x, my_y), (my_x, 1 - my_y), (1 - my_x, 1 - my_y)]
    peer_slots = [(1 - my_x) * 2 + my_y, my_x * 2 + (1 - my_y),
                  (1 - my_x) * 2 + (1 - my_y)]
    r = pl.program_id(0)
    g = pl.program_id(1)
    barrier = pltpu.get_barrier_semaphore()

    def _wrdma(i, src, dst):
        return pltpu.make_async_remote_copy(
            src_ref=src, dst_ref=dst,
            send_sem=w_send.at[i], recv_sem=w_recv.at[i],
            device_id=xnbr, device_id_type=pl.DeviceIdType.MESH)

    def _w_descs():
        descs = [_wrdma(0, c_mine_bf, c_oth_bf)]
        for gg in range(NG):
            descs.append(_wrdma(1 + 2 * gg, wuk_bf.at[gg], wuk_o.at[gg]))
            descs.append(_wrdma(2 + 2 * gg, wuv_bf.at[gg], wuv_o.at[gg]))
        return descs

    @pl.when((r == 0) & (g == 0))
    def _():
        for peer in peers:
            pl.semaphore_signal(barrier, inc=1, device_id=peer,
                                device_id_type=pl.DeviceIdType.MESH)
        pl.semaphore_wait(barrier, 3)
        c_mine_bf[...] = _dot(x_ref[0], wdkv_ref[...]).astype(BF16)
        for gg in range(NG):
            wuk_bf[gg] = wuk_ref[:, gg * GW:(gg + 1) * GW].astype(BF16)
            wuv_bf[gg] = wuv_ref[:, gg * GW:(gg + 1) * GW].astype(BF16)
        for d in _w_descs():
            d.start()
        kr_s[...] = _dot(x_ref[0], wkr_ref[...])

    @pl.when(r == 0)
    def _():
        xm = x_ref[0, pl.ds(p * S4, S4), :]
        q_cache[g] = (_dot(xm, wq_ref[...]) * SCALE).astype(BF16)
        qr_cache[g] = (_dot(xm, wqr_ref[...]) * SCALE).astype(BF16)

    q = q_cache[g, pl.ds(r * SQ, SQ), :]
    qr = qr_cache[g, pl.ds(r * SQ, SQ), :]

    @pl.when((r == 0) & (g == 0))
    def _():
        _wrdma(0, c_mine_bf, c_oth_bf).wait_recv()

    for gg in range(NG):
        @pl.when((r == 0) & (g == gg))
        def _(gg=gg):
            _wrdma(1 + 2 * gg, wuk_bf.at[gg], wuk_o.at[gg]).wait_recv()
            _wrdma(2 + 2 * gg, wuv_bf.at[gg], wuv_o.at[gg]).wait_recv()
            cm = c_mine_bf[...]
            co = c_oth_bf[...]
            k_cache[gg] = (_dot(cm, wuk_bf[gg])
                           + _dot(co, wuk_o[gg])).astype(BF16)
            v_cache[gg] = (_dot(cm, wuv_bf[gg])
                           + _dot(co, wuv_o[gg])).astype(BF16)

    k_g = k_cache[g]
    v_g = v_cache[g]
    kr = kr_s[...].astype(BF16)

    o_cols = []
    for i in range(HG):
        s = _dot_t(q[:, i * DH:(i + 1) * DH],
                   k_g[:, i * DH:(i + 1) * DH])
        s = s + _dot_t(qr[:, i * DR:(i + 1) * DR], kr)
        e = jnp.exp(s)
        denom = jnp.sum(e, axis=1, keepdims=True)
        o_cols.append(
            _dot(e.astype(BF16), v_g[:, i * DH:(i + 1) * DH]) / denom)
    proj = _dot(jnp.concatenate(o_cols, axis=1), wo_ref[...])

    @pl.when(g == 0)
    def _():
        out_acc[r] = proj

    @pl.when(g != 0)
    def _():
        out_acc[r] = out_acc[r] + proj

    for rr in range(NR):
        @pl.when((r == rr) & (g == NG - 1))
        def _(rr=rr):
            my_rows = out_ref.at[0, pl.ds(p * S4 + rr * SQ, SQ), :]
            cp = pltpu.make_async_copy(out_acc.at[rr], my_rows, copy_sem)
            cp.start()
            stage_bf[rr] = out_acc[rr].astype(BF16)
            cp.wait()
            for i, peer in enumerate(peers):
                pltpu.make_async_remote_copy(
                    src_ref=stage_bf.at[rr], dst_ref=recv_bf.at[p, rr],
                    send_sem=ag_send.at[rr * 3 + i],
                    recv_sem=ag_recv.at[p * NR + rr],
                    device_id=peer,
                    device_id_type=pl.DeviceIdType.MESH).start()

    @pl.when((r == NR - 1) & (g == NG - 1))
    def _():
        for d in _w_descs():
            d.wait_send()
        for rr in range(NR):
            for i, peer in enumerate(peers):
                pltpu.make_async_remote_copy(
                    src_ref=stage_bf.at[rr], dst_ref=recv_bf.at[p, rr],
                    send_sem=ag_send.at[rr * 3 + i],
                    recv_sem=ag_recv.at[p * NR + rr],
                    device_id=peer,
                    device_id_type=pl.DeviceIdType.MESH).wait_send()
        for rr in range(NR):
            for i, qp in enumerate(peer_slots):
                pltpu.make_async_remote_copy(
                    src_ref=recv_bf.at[qp, rr], dst_ref=recv_bf.at[qp, rr],
                    send_sem=ag_send.at[rr * 3 + i],
                    recv_sem=ag_recv.at[qp * NR + rr],
                    device_id=(my_x, my_y),
                    device_id_type=pl.DeviceIdType.MESH).wait_recv()
                tmp_f32[...] = recv_bf[qp, rr].astype(F32)
                cp = pltpu.make_async_copy(
                    tmp_f32,
                    out_ref.at[0, pl.ds(qp * S4 + rr * SQ, SQ), :],
                    copy_sem)
                cp.start()
                cp.wait()


def kernel(x, Wdkv, Wuk, Wuv, Wq, Wqr, Wkr, Wo):
    y = pl.pallas_call(
        _body,
        grid=(NR, NG),
        out_shape=jax.ShapeDtypeStruct((1, S, D), F32),
        in_specs=[
            pl.BlockSpec((1, S, D), lambda r, g: (0, 0, 0)),
            pl.BlockSpec((D, DC_SH), lambda r, g: (0, 0)),
            pl.BlockSpec((D, DR), lambda r, g: (0, 0)),
            pl.BlockSpec((DC_SH, D), lambda r, g: (0, 0)),
            pl.BlockSpec((DC_SH, D), lambda r, g: (0, 0)),
            pl.BlockSpec((D, GW),
                         lambda r, g: (0, g * (1 - r) + (NG - 1) * r)),
            pl.BlockSpec((D, HG * DR),
                         lambda r, g: (0, g * (1 - r) + (NG - 1) * r)),
            pl.BlockSpec((GW, D), lambda r, g: (g, 0)),
        ],
        out_specs=pl.BlockSpec(memory_space=pl.ANY),
        scratch_shapes=[
            pltpu.VMEM((S, DC_SH), BF16),
            pltpu.VMEM((S, DR), F32),
            pltpu.VMEM((NG, DC_SH, GW), BF16),
            pltpu.VMEM((NG, DC_SH, GW), BF16),
            pltpu.VMEM((S, DC_SH), BF16),
            pltpu.VMEM((NG, DC_SH, GW), BF16),
            pltpu.VMEM((NG, DC_SH, GW), BF16),
            pltpu.VMEM((NG, S, GW), BF16),
            pltpu.VMEM((NG, S, GW), BF16),
            pltpu.VMEM((NG, S4, GW), BF16),
            pltpu.VMEM((NG, S4, HG * DR), BF16),
            pltpu.VMEM((NR, SQ, D), F32),
            pltpu.VMEM((NR, SQ, D), BF16),
            pltpu.VMEM((4, NR, SQ, D), BF16),
            pltpu.VMEM((SQ, D), F32),
            pltpu.SemaphoreType.DMA((9,)),
            pltpu.SemaphoreType.DMA((9,)),
            pltpu.SemaphoreType.DMA,
            pltpu.SemaphoreType.DMA((NR * 3,)),
            pltpu.SemaphoreType.DMA((4 * NR,)),
        ],
        compiler_params=pltpu.CompilerParams(
            collective_id=0, vmem_limit_bytes=62 * 1024 * 1024,
        ),
    )(x, Wdkv, Wkr, Wuk, Wuv, Wq, Wqr, Wo)

    return y
